# baseline (speedup 1.0000x reference)
import functools

import jax
import jax.numpy as jnp
from jax import lax
from jax.experimental import pallas as pl
from jax.experimental.pallas import tpu as pltpu
from jax.experimental.pallas import tpu_sc as plsc


_NC, _NS = 2, 16
_NW = _NC * _NS
_CHUNK = 8
_RREP = 2


@functools.lru_cache(maxsize=None)
def _make_sc_kernel(S, B, V, H):
    HT = H // 128
    ST = S // 128
    CS = 2 * ST
    GS = B * CS
    sg_per_b = _NW // B
    s_per_w = S // sg_per_b
    n_ch = s_per_w // _CHUNK
    i_per_w = S // _NW
    nb = i_per_w * B
    GM = B * ST
    n_bulk = i_per_w // _RREP
    n_diag = i_per_w * B
    mesh = plsc.VectorSubcoreMesh(core_axis_name="c", subcore_axis_name="s")

    @functools.partial(
        pl.kernel,
        mesh=mesh,
        out_type=[
            jax.ShapeDtypeStruct((S, HT, B, 128), jnp.float32),
            jax.ShapeDtypeStruct((S * GS, 128), jnp.float32),
            jax.ShapeDtypeStruct((S * GM, 128), jnp.float32),
        ],
        scratch_types=[
            pltpu.VMEM((s_per_w,), jnp.int32),
            pltpu.VMEM((nb,), jnp.int32),
            pltpu.VMEM((n_diag,), jnp.int32),
            pltpu.VMEM((2 * GS, 128), jnp.float32),
            pltpu.VMEM((_RREP * GM, 128), jnp.float32),
            pltpu.VMEM((n_diag, 128), jnp.float32),
            pltpu.VMEM((_CHUNK, H), jnp.float32),
            pltpu.VMEM((_CHUNK, H), jnp.float32),
            pltpu.SemaphoreType.DMA,
            pltpu.SemaphoreType.DMA,
            pltpu.SemaphoreType.DMA,
            pltpu.SemaphoreType.DMA,
            pltpu.SemaphoreType.DMA,
            pltpu.SemaphoreType.DMA,
            pltpu.SemaphoreType.DMA,
        ],
    )
    def sc_kernel(idx_hbm, pstack_hbm, bits_hbm, pat_hbm, diag_hbm, table_hbm,
                  tok_hbm, seg_hbm, mask_hbm,
                  idx_v, bits_v, didx_v, pstack_v, rbuf, dbuf, rows0, rows1,
                  ga, gb, sa, sb, qsem, msem, dsem):
        wid = lax.axis_index("s") * _NC + lax.axis_index("c")
        b = wid // sg_per_b
        s0 = (wid % sg_per_b) * s_per_w
        i0 = wid * i_per_w
        pltpu.sync_copy(idx_hbm.at[pl.ds(wid * s_per_w, s_per_w)], idx_v)
        pltpu.sync_copy(bits_hbm.at[pl.ds(i0 * B, nb)], bits_v)
        pltpu.sync_copy(pstack_hbm, pstack_v)
        for r in range(_RREP):
            pltpu.sync_copy(pat_hbm, rbuf.at[pl.ds(r * GM, GM), :])
        for bb4 in range(B):
            pltpu.sync_copy(
                diag_hbm.at[pl.ds(bb4 * S + i0, i_per_w), :],
                dbuf.at[pl.ds(bb4 * i_per_w, i_per_w), :])

        bulk = [
            pltpu.async_copy(
                rbuf,
                mask_hbm.at[pl.ds((i0 + t * _RREP) * GM, _RREP * GM), :],
                msem)
            for t in range(n_bulk)
        ]
        lane = lax.iota(jnp.int32, 16)
        for t in range(n_diag // 16):
            kvec = t * 16 + lane
            ivec = i0 + (kvec & (i_per_w - 1))
            bvec = kvec >> (i_per_w.bit_length() - 1)
            didx_v[pl.ds(t * 16, 16)] = ivec * GM + bvec * ST + (ivec >> 7)

        seg_cps = []
        for t in range(nb // 16):
            bv = bits_v[pl.ds(t * 16, 16)]
            for j in range(16):
                kb = t * 16 + j
                iloc, bb = kb >> 2, kb & 3
                pol = bv[j]
                seg_cps.append(pltpu.async_copy(
                    pstack_v.at[pl.ds(pol * GS + bb * CS, CS), :],
                    seg_hbm.at[pl.ds((i0 + iloc) * GS + bb * CS, CS), :],
                    qsem))

        bufs = (rows0, rows1)
        gsems = (ga, gb)
        ssems = (sa, sb)

        def gather(c):
            return pltpu.async_copy(
                table_hbm.at[idx_v.at[pl.ds(c * _CHUNK, _CHUNK)]],
                bufs[c & 1], gsems[c & 1])

        def put(c):
            return [
                pltpu.async_copy(
                    bufs[c & 1].at[:, pl.ds(ht * 128, 128)],
                    tok_hbm.at[pl.ds(s0 + c * _CHUNK, _CHUNK), ht, b, :],
                    ssems[c & 1])
                for ht in range(HT)
            ]

        def drain(copies):
            for cp in copies:
                cp.wait()

        g = [None] * n_ch
        st = [None] * n_ch
        g[0] = gather(0)
        for c in range(n_ch):
            if c + 1 < n_ch:
                if c >= 1:
                    drain(st[c - 1])
                g[c + 1] = gather(c + 1)
            g[c].wait()
            st[c] = put(c)

        drain(bulk)
        dcp = pltpu.async_copy(dbuf, mask_hbm.at[didx_v], dsem)
        if n_ch >= 2:
            drain(st[n_ch - 2])
        drain(st[n_ch - 1])
        drain(seg_cps)
        dcp.wait()

    return sc_kernel


@functools.lru_cache(maxsize=None)
def _make_tc_prep(S, B):
    ST = S // 128

    def body(am_ref, pat_ref, diag_ref):
        jt, b = pl.program_id(0), pl.program_id(1)
        am = am_ref[b, jt, :][None, :]
        pat_ref[pl.ds(b * ST + jt, 1), :] = (am > 0.0).astype(jnp.float32)
        row = lax.broadcasted_iota(jnp.int32, (128, 128), 0)
        col = lax.broadcasted_iota(jnp.int32, (128, 128), 1)
        eye = (row == col).astype(jnp.float32)
        diag_ref[...] = (jnp.broadcast_to(am, (128, 128)) > eye
                         ).astype(jnp.float32)

    return pl.pallas_call(
        body,
        grid=(ST, B),
        in_specs=[
            pl.BlockSpec((B, ST, 128), lambda jt, b: (0, 0, 0)),
        ],
        out_specs=[
            pl.BlockSpec((B * ST, 128), lambda jt, b: (0, 0)),
            pl.BlockSpec((128, 128), lambda jt, b: (b * ST + jt, 0)),
        ],
        out_shape=[
            jax.ShapeDtypeStruct((B * ST, 128), jnp.float32),
            jax.ShapeDtypeStruct((B * S, 128), jnp.float32),
        ],
    )


@functools.lru_cache(maxsize=None)
def _make_tc_dense(S, B, H, BR):
    G = S // BR
    ST = S // 128
    HT = H // 128
    GM = B * ST

    def body(pos_ref, if_ref, posout_ref):
        arg = pos_ref[...][:, :, None] * if_ref[...]
        half = HT // 2
        val = jnp.concatenate(
            [jnp.sin(arg[:, :half, :]), jnp.cos(arg[:, half:, :])], axis=1)
        posout_ref[...] = jnp.broadcast_to(
            val[:, :, None, :], (2 * BR, HT, B, 128))

    return pl.pallas_call(
        body,
        grid=(G,),
        in_specs=[
            pl.BlockSpec((2 * BR, 1), lambda i: (i, 0)),
            pl.BlockSpec((1, HT, 128), lambda i: (0, 0, 0)),
        ],
        out_specs=[
            pl.BlockSpec((2 * BR, HT, B, 128), lambda i: (i, 0, 0, 0)),
        ],
        out_shape=[
            jax.ShapeDtypeStruct((2 * S, HT, B, 128), jnp.float32),
        ],
    )


def kernel(token_ids, segment_ids, attn_mask, token_embeddings):
    B, S = token_ids.shape
    V, H = token_embeddings.shape
    ST = S // 128
    HT = H // 128
    GS = B * 2 * ST

    idx = token_ids.reshape(-1)
    am3 = attn_mask.reshape(B, ST, 128)
    seg32 = segment_ids.astype(jnp.int32)
    bits = seg32.T.reshape(-1)
    q3 = (seg32.reshape(B, ST, 1, 128)
          ^ jnp.arange(2, dtype=jnp.int32)[None, None, :, None]
          ).reshape(GS, 128)
    pstack = jnp.concatenate([1 ^ q3, q3], axis=0
                             ).astype(jnp.float32)
    pos_col = jnp.arange(S, -S, -1.0, dtype=jnp.float32)[:, None]
    freq_seq = jnp.arange(0, H, 2.0, dtype=jnp.float32)
    inv_freq = 1.0 / jnp.power(10000.0, freq_seq / H)
    ifu = jnp.concatenate([inv_freq, inv_freq]).reshape(1, HT, 128)

    pat, diag = _make_tc_prep(S, B)(am3)
    tok4, seg2, mask2 = _make_sc_kernel(S, B, V, H)(idx, pstack, bits, pat,
                                                    diag, token_embeddings)
    (pos4,) = _make_tc_dense(S, B, H, 32)(pos_col, ifu)

    token_embed = tok4.transpose(0, 2, 1, 3).reshape(S, B, H)
    segment_embed = (seg2.reshape(S, B, ST, 2, 128)
                     .transpose(0, 2, 4, 1, 3).reshape(S, S, B, 2))
    pos_embed = pos4.transpose(0, 2, 1, 3).reshape(2 * S, B, H)
    non_target_mask = (mask2.reshape(S, B, ST, 128)
                       .transpose(0, 2, 3, 1).reshape(S, S, B, 1))
    return (token_embed, segment_embed, pos_embed, non_target_mask)


# speedup vs baseline: 1.1081x; 1.1081x over previous
import functools

import jax
import jax.numpy as jnp
from jax import lax
from jax.experimental import pallas as pl
from jax.experimental.pallas import tpu as pltpu
from jax.experimental.pallas import tpu_sc as plsc


_NC, _NS = 2, 16
_NW = _NC * _NS
_CHUNK = 16


@functools.lru_cache(maxsize=None)
def _make_sc_kernel(S, B, V, H):
    HT = H // 128
    ST = S // 128
    CS = 2 * ST
    GS = B * CS
    sg_per_b = _NW // B
    s_per_w = S // sg_per_b
    n_ch = s_per_w // _CHUNK
    i_per_w = S // _NW
    nb = i_per_w * B
    mesh = plsc.VectorSubcoreMesh(core_axis_name="c", subcore_axis_name="s")

    @functools.partial(
        pl.kernel,
        mesh=mesh,
        out_type=[
            jax.ShapeDtypeStruct((S, HT, B, 128), jnp.float32),
            jax.ShapeDtypeStruct((S * GS, 128), jnp.float32),
        ],
        scratch_types=[
            pltpu.VMEM((s_per_w,), jnp.int32),
            pltpu.VMEM((nb,), jnp.int32),
            pltpu.VMEM((2 * GS, 128), jnp.float32),
            pltpu.VMEM((_CHUNK, H), jnp.float32),
            pltpu.VMEM((_CHUNK, H), jnp.float32),
            pltpu.SemaphoreType.DMA,
            pltpu.SemaphoreType.DMA,
            pltpu.SemaphoreType.DMA,
            pltpu.SemaphoreType.DMA,
            pltpu.SemaphoreType.DMA,
        ],
    )
    def sc_kernel(idx_hbm, pstack_hbm, bits_hbm, table_hbm, tok_hbm, seg_hbm,
                  idx_v, bits_v, pstack_v, rows0, rows1,
                  ga, gb, sa, sb, qsem):
        wid = lax.axis_index("s") * _NC + lax.axis_index("c")
        b = wid // sg_per_b
        s0 = (wid % sg_per_b) * s_per_w
        i0 = wid * i_per_w
        pltpu.sync_copy(idx_hbm.at[pl.ds(wid * s_per_w, s_per_w)], idx_v)
        pltpu.sync_copy(bits_hbm.at[pl.ds(i0 * B, nb)], bits_v)
        pltpu.sync_copy(pstack_hbm, pstack_v)

        seg_cps = []
        for t in range(nb // 16):
            bv = bits_v[pl.ds(t * 16, 16)]
            for j in range(16):
                kb = t * 16 + j
                iloc, bb = kb >> 2, kb & 3
                pol = bv[j]
                seg_cps.append(pltpu.async_copy(
                    pstack_v.at[pl.ds(pol * GS + bb * CS, CS), :],
                    seg_hbm.at[pl.ds((i0 + iloc) * GS + bb * CS, CS), :],
                    qsem))

        bufs = (rows0, rows1)
        gsems = (ga, gb)
        ssems = (sa, sb)

        def gather(c):
            return pltpu.async_copy(
                table_hbm.at[idx_v.at[pl.ds(c * _CHUNK, _CHUNK)]],
                bufs[c & 1], gsems[c & 1])

        def put(c):
            return [
                pltpu.async_copy(
                    bufs[c & 1].at[:, pl.ds(ht * 128, 128)],
                    tok_hbm.at[pl.ds(s0 + c * _CHUNK, _CHUNK), ht, b, :],
                    ssems[c & 1])
                for ht in range(HT)
            ]

        def drain(copies):
            for cp in copies:
                cp.wait()

        g = [None] * n_ch
        st = [None] * n_ch
        g[0] = gather(0)
        for c in range(n_ch):
            if c + 1 < n_ch:
                if c >= 1:
                    drain(st[c - 1])
                g[c + 1] = gather(c + 1)
            g[c].wait()
            st[c] = put(c)

        if n_ch >= 2:
            drain(st[n_ch - 2])
        drain(st[n_ch - 1])
        drain(seg_cps)

    return sc_kernel


@functools.lru_cache(maxsize=None)
def _make_tc_dense(S, B, H, BR):
    G = S // BR
    ST = S // 128
    HT = H // 128
    GM = B * ST

    def body(am_ref, pos_ref, if_ref, posout_ref, mask_ref):
        i = pl.program_id(0)

        gm = lax.broadcasted_iota(jnp.int32, (BR, GM, 128), 1)
        mm = lax.broadcasted_iota(jnp.int32, (BR, GM, 128), 2)
        jvec = ((gm & (ST - 1)) << 7) + mm
        ivec = lax.broadcasted_iota(jnp.int32, (BR, GM, 128), 0) + i * BR
        eye = (jvec == ivec).astype(jnp.float32)
        mask_ref[...] = ((am_ref[...] - eye) > 0).astype(jnp.float32)

        arg = pos_ref[...][:, :, None] * if_ref[...]
        half = HT // 2
        val = jnp.concatenate(
            [jnp.sin(arg[:, :half, :]), jnp.cos(arg[:, half:, :])], axis=1)
        posout_ref[...] = jnp.broadcast_to(
            val[:, :, None, :], (2 * BR, HT, B, 128))

    return pl.pallas_call(
        body,
        grid=(G,),
        in_specs=[
            pl.BlockSpec((1, GM, 128), lambda i: (0, 0, 0)),
            pl.BlockSpec((2 * BR, 1), lambda i: (i, 0)),
            pl.BlockSpec((1, HT, 128), lambda i: (0, 0, 0)),
        ],
        out_specs=[
            pl.BlockSpec((2 * BR, HT, B, 128), lambda i: (i, 0, 0, 0)),
            pl.BlockSpec((BR, GM, 128), lambda i: (i, 0, 0)),
        ],
        out_shape=[
            jax.ShapeDtypeStruct((2 * S, HT, B, 128), jnp.float32),
            jax.ShapeDtypeStruct((S, GM, 128), jnp.float32),
        ],
    )


def kernel(token_ids, segment_ids, attn_mask, token_embeddings):
    B, S = token_ids.shape
    V, H = token_embeddings.shape
    ST = S // 128
    HT = H // 128
    GS = B * 2 * ST

    idx = token_ids.reshape(-1)
    am3 = attn_mask.reshape(1, B * ST, 128)
    seg32 = segment_ids.astype(jnp.int32)
    bits = seg32.T.reshape(-1)
    q3 = (seg32.reshape(B, ST, 1, 128)
          ^ jnp.arange(2, dtype=jnp.int32)[None, None, :, None]
          ).reshape(GS, 128)
    pstack = jnp.concatenate([1 ^ q3, q3], axis=0
                             ).astype(jnp.float32)
    pos_col = jnp.arange(S, -S, -1.0, dtype=jnp.float32)[:, None]
    freq_seq = jnp.arange(0, H, 2.0, dtype=jnp.float32)
    inv_freq = 1.0 / jnp.power(10000.0, freq_seq / H)
    ifu = jnp.concatenate([inv_freq, inv_freq]).reshape(1, HT, 128)

    tok4, seg2 = _make_sc_kernel(S, B, V, H)(idx, pstack, bits,
                                             token_embeddings)
    pos4, mask3 = _make_tc_dense(S, B, H, 64)(am3, pos_col, ifu)

    token_embed = tok4.transpose(0, 2, 1, 3).reshape(S, B, H)
    segment_embed = (seg2.reshape(S, B, ST, 2, 128)
                     .transpose(0, 2, 4, 1, 3).reshape(S, S, B, 2))
    pos_embed = pos4.transpose(0, 2, 1, 3).reshape(2 * S, B, H)
    non_target_mask = (mask3.reshape(S, B, ST, 128)
                       .transpose(0, 2, 3, 1).reshape(S, S, B, 1))
    return (token_embed, segment_embed, pos_embed, non_target_mask)


# speedup vs baseline: 1.1131x; 1.0045x over previous
import functools

import jax
import jax.numpy as jnp
from jax import lax
from jax.experimental import pallas as pl
from jax.experimental.pallas import tpu as pltpu
from jax.experimental.pallas import tpu_sc as plsc


_NC, _NS = 2, 16
_NW = _NC * _NS
_CHUNK = 16


@functools.lru_cache(maxsize=None)
def _make_sc_kernel(S, B, V, H):
    HT = H // 128
    ST = S // 128
    CS = 2 * ST
    GS = B * CS
    sg_per_b = _NW // B
    s_per_w = S // sg_per_b
    n_ch = s_per_w // _CHUNK
    i_per_w = S // _NW
    nb = i_per_w * B
    mesh = plsc.VectorSubcoreMesh(core_axis_name="c", subcore_axis_name="s")

    @functools.partial(
        pl.kernel,
        mesh=mesh,
        out_type=[
            jax.ShapeDtypeStruct((S, HT, B, 128), jnp.float32),
            jax.ShapeDtypeStruct((S * GS, 128), jnp.float32),
        ],
        scratch_types=[
            pltpu.VMEM((s_per_w,), jnp.int32),
            pltpu.VMEM((nb,), jnp.int32),
            pltpu.VMEM((2 * GS, 128), jnp.float32),
            pltpu.VMEM((_CHUNK, H), jnp.float32),
            pltpu.VMEM((_CHUNK, H), jnp.float32),
            pltpu.SemaphoreType.DMA,
            pltpu.SemaphoreType.DMA,
            pltpu.SemaphoreType.DMA,
            pltpu.SemaphoreType.DMA,
            pltpu.SemaphoreType.DMA,
        ],
    )
    def sc_kernel(idx_hbm, pstack_hbm, bits_hbm, table_hbm, tok_hbm, seg_hbm,
                  idx_v, bits_v, pstack_v, rows0, rows1,
                  ga, gb, sa, sb, qsem):
        wid = lax.axis_index("s") * _NC + lax.axis_index("c")
        b = wid // sg_per_b
        s0 = (wid % sg_per_b) * s_per_w
        i0 = wid * i_per_w
        pltpu.sync_copy(idx_hbm.at[pl.ds(wid * s_per_w, s_per_w)], idx_v)
        pltpu.sync_copy(bits_hbm.at[pl.ds(i0 * B, nb)], bits_v)
        pltpu.sync_copy(pstack_hbm, pstack_v)

        seg_cps = []
        for t in range(nb // 16):
            bv = bits_v[pl.ds(t * 16, 16)]
            for j in range(16):
                kb = t * 16 + j
                iloc, bb = kb >> 2, kb & 3
                pol = bv[j]
                seg_cps.append(pltpu.async_copy(
                    pstack_v.at[pl.ds(pol * GS + bb * CS, CS), :],
                    seg_hbm.at[pl.ds((i0 + iloc) * GS + bb * CS, CS), :],
                    qsem))

        bufs = (rows0, rows1)
        gsems = (ga, gb)
        ssems = (sa, sb)

        def gather(c):
            return pltpu.async_copy(
                table_hbm.at[idx_v.at[pl.ds(c * _CHUNK, _CHUNK)]],
                bufs[c & 1], gsems[c & 1])

        def put(c):
            return [
                pltpu.async_copy(
                    bufs[c & 1].at[:, pl.ds(ht * 128, 128)],
                    tok_hbm.at[pl.ds(s0 + c * _CHUNK, _CHUNK), ht, b, :],
                    ssems[c & 1])
                for ht in range(HT)
            ]

        def drain(copies):
            for cp in copies:
                cp.wait()

        g = [None] * n_ch
        st = [None] * n_ch
        g[0] = gather(0)
        for c in range(n_ch):
            if c + 1 < n_ch:
                if c >= 1:
                    drain(st[c - 1])
                g[c + 1] = gather(c + 1)
            g[c].wait()
            st[c] = put(c)

        if n_ch >= 2:
            drain(st[n_ch - 2])
        drain(st[n_ch - 1])
        drain(seg_cps)

    return sc_kernel


@functools.lru_cache(maxsize=None)
def _make_tc_dense(S, B, H, BR):
    G = S // BR
    ST = S // 128
    HT = H // 128
    GM = B * ST

    def body(am_ref, pos_ref, if_ref, posout_ref, mask_ref):
        i = pl.program_id(0)

        gm = lax.broadcasted_iota(jnp.int32, (BR, GM, 128), 1)
        mm = lax.broadcasted_iota(jnp.int32, (BR, GM, 128), 2)
        jvec = ((gm & (ST - 1)) << 7) + mm
        ivec = lax.broadcasted_iota(jnp.int32, (BR, GM, 128), 0) + i * BR
        eye = (jvec == ivec).astype(jnp.float32)
        mask_ref[...] = ((am_ref[...] - eye) > 0).astype(jnp.float32)

        arg = pos_ref[...][:, :, None] * if_ref[...]
        half = HT // 2
        val = jnp.concatenate(
            [jnp.sin(arg[:, :half, :]), jnp.cos(arg[:, half:, :])], axis=1)
        posout_ref[...] = jnp.broadcast_to(
            val[:, :, None, :], (2 * BR, HT, B, 128))

    return pl.pallas_call(
        body,
        grid=(G,),
        in_specs=[
            pl.BlockSpec((1, GM, 128), lambda i: (0, 0, 0)),
            pl.BlockSpec((2 * BR, 1), lambda i: (i, 0)),
            pl.BlockSpec((1, HT, 128), lambda i: (0, 0, 0)),
        ],
        out_specs=[
            pl.BlockSpec((2 * BR, HT, B, 128), lambda i: (i, 0, 0, 0)),
            pl.BlockSpec((BR, GM, 128), lambda i: (i, 0, 0)),
        ],
        out_shape=[
            jax.ShapeDtypeStruct((2 * S, HT, B, 128), jnp.float32),
            jax.ShapeDtypeStruct((S, GM, 128), jnp.float32),
        ],
    )


def kernel(token_ids, segment_ids, attn_mask, token_embeddings):
    B, S = token_ids.shape
    V, H = token_embeddings.shape
    ST = S // 128
    HT = H // 128
    GS = B * 2 * ST

    idx = token_ids.reshape(-1)
    am3 = attn_mask.reshape(1, B * ST, 128)
    seg32 = segment_ids.astype(jnp.int32)
    bits = seg32.T.reshape(-1)
    q3 = (seg32.reshape(B, ST, 1, 128)
          ^ jnp.arange(2, dtype=jnp.int32)[None, None, :, None]
          ).reshape(GS, 128)
    pstack = jnp.concatenate([1 ^ q3, q3], axis=0
                             ).astype(jnp.float32)
    pos_col = jnp.arange(S, -S, -1.0, dtype=jnp.float32)[:, None]
    freq_seq = jnp.arange(0, H, 2.0, dtype=jnp.float32)
    inv_freq = 1.0 / jnp.power(10000.0, freq_seq / H)
    ifu = jnp.concatenate([inv_freq, inv_freq]).reshape(1, HT, 128)

    tok4, seg2 = _make_sc_kernel(S, B, V, H)(idx, pstack, bits,
                                             token_embeddings)
    pos4, mask3 = _make_tc_dense(S, B, H, 128)(am3, pos_col, ifu)

    token_embed = tok4.transpose(0, 2, 1, 3).reshape(S, B, H)
    segment_embed = (seg2.reshape(S, B, ST, 2, 128)
                     .transpose(0, 2, 4, 1, 3).reshape(S, S, B, 2))
    pos_embed = pos4.transpose(0, 2, 1, 3).reshape(2 * S, B, H)
    non_target_mask = (mask3.reshape(S, B, ST, 128)
                       .transpose(0, 2, 3, 1).reshape(S, S, B, 1))
    return (token_embed, segment_embed, pos_embed, non_target_mask)
